# baseline (speedup 1.0000x reference)
import functools

import jax
import jax.numpy as jnp
from jax import lax
from jax.experimental import pallas as pl
from jax.experimental.pallas import tpu as pltpu
from jax.experimental.pallas import tpu_sc as plsc


def _fps_body(xs_ref, ys_ref, zs_ref, ox_ref, oy_ref, oz_ref, d_ref, *, npoint):
  B, N = xs_ref.shape
  xs = xs_ref[...]
  ys = ys_ref[...]
  zs = zs_ref[...]
  lane = lax.broadcasted_iota(jnp.int32, (B, N), 1)
  col = lax.broadcasted_iota(jnp.int32, (B, npoint), 1)
  d_ref[...] = jnp.full((B, N), 1e10, jnp.float32)
  ox_ref[...] = jnp.zeros((B, npoint), jnp.float32)
  oy_ref[...] = jnp.zeros((B, npoint), jnp.float32)
  oz_ref[...] = jnp.zeros((B, npoint), jnp.float32)

  def step(i, far):
    oh = (lane == far).astype(jnp.float32)
    cx = jnp.sum(xs * oh, axis=1, keepdims=True)
    cy = jnp.sum(ys * oh, axis=1, keepdims=True)
    cz = jnp.sum(zs * oh, axis=1, keepdims=True)
    coh = (col == i).astype(jnp.float32)
    ox_ref[...] += cx * coh
    oy_ref[...] += cy * coh
    oz_ref[...] += cz * coh
    dx = xs - cx
    dy = ys - cy
    dz = zs - cz
    d = dx * dx + dy * dy + dz * dz
    dm = jnp.minimum(d_ref[...], d)
    d_ref[...] = dm
    m = jnp.max(dm, axis=1, keepdims=True)
    far2 = jnp.min(jnp.where(dm == m, lane, N), axis=1, keepdims=True)
    return far2

  lax.fori_loop(0, npoint, step, jnp.zeros((B, 1), jnp.int32))


def _fps(xs, ys, zs, npoint):
  B, N = xs.shape
  out = jax.ShapeDtypeStruct((B, npoint), jnp.float32)
  return pl.pallas_call(
      functools.partial(_fps_body, npoint=npoint),
      out_shape=[out, out, out],
      scratch_shapes=[pltpu.VMEM((B, N), jnp.float32)],
  )(xs, ys, zs)


def _ball_group(xs, ys, zs, cx, cy, cz, radius, nsample, base_stride=None):
  B, N = xs.shape
  S = cx.shape[1]
  NW = 32
  WPB = NW // B
  SPW = S // WPB
  NCH = N // 16
  KCH = nsample // 16
  r2 = radius * radius
  want_idx = base_stride is not None

  out_type = [jax.ShapeDtypeStruct((B, S * nsample), jnp.float32)] * 3
  if want_idx:
    out_type.append(jax.ShapeDtypeStruct((B, S * nsample), jnp.int32))
  scratch = (
      [pltpu.VMEM((N,), jnp.float32)] * 3
      + [pltpu.VMEM((SPW,), jnp.float32)] * 3
      + [pltpu.VMEM((SPW * nsample,), jnp.float32)] * 3
      + [pltpu.VMEM((nsample,), jnp.int32), pltpu.VMEM((16,), jnp.int32),
         pltpu.VMEM((16,), jnp.int32)]
  )
  if want_idx:
    scratch.append(pltpu.VMEM((SPW * nsample,), jnp.int32))

  mesh = plsc.VectorSubcoreMesh(core_axis_name="c", subcore_axis_name="s")

  @functools.partial(
      pl.kernel, mesh=mesh, out_type=out_type, scratch_types=scratch,
      compiler_params=pltpu.CompilerParams(needs_layout_passes=False))
  def k(xs_h, ys_h, zs_h, cx_h, cy_h, cz_h, *rest):
    if want_idx:
      (ox_h, oy_h, oz_h, oi_h, xv, yv, zv, cxv, cyv, czv,
       gxb, gyb, gzb, idxb, cntb, zb, gib) = rest
    else:
      (ox_h, oy_h, oz_h, xv, yv, zv, cxv, cyv, czv,
       gxb, gyb, gzb, idxb, cntb, zb) = rest
    w = lax.axis_index("s") * 2 + lax.axis_index("c")
    b = w // WPB
    off = (w % WPB) * SPW
    pltpu.sync_copy(xs_h.at[b], xv)
    pltpu.sync_copy(ys_h.at[b], yv)
    pltpu.sync_copy(zs_h.at[b], zv)
    pltpu.sync_copy(cx_h.at[b, pl.ds(off, SPW)], cxv)
    pltpu.sync_copy(cy_h.at[b, pl.ds(off, SPW)], cyv)
    pltpu.sync_copy(cz_h.at[b, pl.ds(off, SPW)], czv)
    zb[pl.ds(0, 16)] = jnp.zeros((16,), jnp.int32)

    def per_cchunk(ci, carry):
      for k in range(16):
        sidx = ci * 16 + k

        def chunk(j, c2):
          done = cntb[pl.ds(0, 16)][0] >= nsample

          @pl.when(jnp.logical_not(done))
          def _():
            base = pl.multiple_of(j * 16, 16)
            sp = jnp.full((16,), sidx, jnp.int32)
            c_xv = plsc.load_gather(cxv, [sp])
            c_yv = plsc.load_gather(cyv, [sp])
            c_zv = plsc.load_gather(czv, [sp])
            px = xv[pl.ds(base, 16)]
            py = yv[pl.ds(base, 16)]
            pz = zv[pl.ds(base, 16)]
            dx = px - c_xv
            dy = py - c_yv
            dz = pz - c_zv
            d2 = dx * dx + dy * dy + dz * dz
            m = d2 < r2
            mi = jnp.where(m, 1, 0)
            cnt_v = cntb[pl.ds(0, 16)]
            pos = cnt_v + plsc.cumsum(mi) - 1
            valid = jnp.logical_and(m, pos < nsample)
            iv = lax.iota(jnp.int32, 16) + base
            plsc.store_scatter(idxb, [pos], iv, mask=valid)
            cntb[pl.ds(0, 16)] = cnt_v + plsc.all_reduce_population_count(m)
          return c2

        cntb[pl.ds(0, 16)] = jnp.zeros((16,), jnp.int32)
        lax.fori_loop(0, NCH, chunk, 0)
        cnt_v = jnp.minimum(cntb[pl.ds(0, 16)], nsample)
        sp = jnp.full((16,), sidx, jnp.int32)
        c_xv = plsc.load_gather(cxv, [sp])
        c_yv = plsc.load_gather(cyv, [sp])
        c_zv = plsc.load_gather(czv, [sp])
        first = plsc.load_gather(idxb, [zb[pl.ds(0, 16)]])
        for kk in range(KCH):
          kb = kk * 16
          lanes = lax.iota(jnp.int32, 16) + kb
          v = idxb[pl.ds(kb, 16)]
          v = jnp.where(lanes < cnt_v, v, first)
          gx = plsc.load_gather(xv, [v]) - c_xv
          gy = plsc.load_gather(yv, [v]) - c_yv
          gz = plsc.load_gather(zv, [v]) - c_zv
          o = pl.multiple_of(sidx * nsample + kb, 16)
          gxb[pl.ds(o, 16)] = gx
          gyb[pl.ds(o, 16)] = gy
          gzb[pl.ds(o, 16)] = gz
          if want_idx:
            gib[pl.ds(o, 16)] = v + b * base_stride
      return carry

    lax.fori_loop(0, SPW // 16, per_cchunk, 0)
    dst = pl.ds(off * nsample, SPW * nsample)
    pltpu.sync_copy(gxb, ox_h.at[b, dst])
    pltpu.sync_copy(gyb, oy_h.at[b, dst])
    pltpu.sync_copy(gzb, oz_h.at[b, dst])
    if want_idx:
      pltpu.sync_copy(gib, oi_h.at[b, dst])

  return k(xs, ys, zs, cx, cy, cz)


def _row_gather(table, idx):
  V, D = table.shape
  R = idx.shape[0]
  NW = 32
  RPW = R // NW
  CH = 128
  NC = RPW // CH
  mesh = plsc.VectorSubcoreMesh(core_axis_name="c", subcore_axis_name="s")

  @functools.partial(
      pl.kernel, mesh=mesh,
      compiler_params=pltpu.CompilerParams(needs_layout_passes=False),
      out_type=jax.ShapeDtypeStruct((R, D), jnp.float32),
      scratch_types=[
          pltpu.VMEM((CH,), jnp.int32),
          pltpu.VMEM((CH, D), jnp.float32),
          pltpu.SemaphoreType.DMA,
      ],
  )
  def k(tab_h, idx_h, out_h, idxv, rowsv, sem):
    w = lax.axis_index("s") * 2 + lax.axis_index("c")
    basew = w * RPW

    def chunk(i, carry):
      o = basew + i * CH
      pltpu.sync_copy(idx_h.at[pl.ds(o, CH)], idxv)
      pltpu.async_copy(tab_h.at[idxv], rowsv, sem).wait()
      pltpu.sync_copy(rowsv, out_h.at[pl.ds(o, CH)])
      return carry

    lax.fori_loop(0, NC, chunk, 0)

  return k(table, idx)


def _b16(x):
  return x.astype(jnp.bfloat16).astype(jnp.float32)


def _stats_update(st_ref, y, is_first):
  @pl.when(is_first)
  def _():
    st_ref[...] = jnp.zeros_like(st_ref)
  s = jnp.sum(y, axis=0, keepdims=True)
  q = jnp.sum(y * y, axis=0, keepdims=True)
  pad = jnp.zeros((6, y.shape[1]), jnp.float32)
  st_ref[...] += jnp.concatenate([s, q, pad], axis=0)


def _bn_relu(x, st_ref, gb_ref, inv_n):
  mean = st_ref[0:1, :] * inv_n
  var = st_ref[1:2, :] * inv_n - mean * mean
  sc = gb_ref[0:1, :] * lax.rsqrt(var + 1e-5)
  sh = gb_ref[1:2, :] - mean * sc
  return jnp.maximum(x * sc + sh, 0.0)


def _l1_xyz_body(gx_ref, gy_ref, gz_ref, w_ref, y_ref, st_ref):
  y = (_b16(gx_ref[...]) * _b16(w_ref[0:1, :])
       + _b16(gy_ref[...]) * _b16(w_ref[1:2, :])
       + _b16(gz_ref[...]) * _b16(w_ref[2:3, :]))
  y_ref[...] = y
  _stats_update(st_ref, y, pl.program_id(0) == 0)


def _l1_feat_body(gx_ref, gy_ref, gz_ref, f_ref, w_ref, wf_ref, y_ref, st_ref):
  y = jnp.dot(f_ref[...], wf_ref[...], preferred_element_type=jnp.float32)
  y += (_b16(gx_ref[...]) * _b16(w_ref[0:1, :])
        + _b16(gy_ref[...]) * _b16(w_ref[1:2, :])
        + _b16(gz_ref[...]) * _b16(w_ref[2:3, :]))
  y_ref[...] = y
  _stats_update(st_ref, y, pl.program_id(0) == 0)


def _mid_body(x_ref, st_ref, gb_ref, wf_ref, y_ref, sto_ref, *, inv_n):
  z = _bn_relu(x_ref[...], st_ref, gb_ref, inv_n)
  y = jnp.dot(z, wf_ref[...], preferred_element_type=jnp.float32)
  y_ref[...] = y
  _stats_update(sto_ref, y, pl.program_id(0) == 0)


def _pool_body(x_ref, st_ref, gb_ref, o_ref, *, inv_n, g):
  R, C = x_ref.shape
  z = _bn_relu(x_ref[...], st_ref, gb_ref, inv_n)
  o_ref[...] = jnp.max(z.reshape(R // g, g, C), axis=1)


def _pad8(rows):
  c = rows[0].shape[0]
  out = jnp.zeros((8, c), jnp.float32)
  for i, r in enumerate(rows):
    out = out.at[i].set(r)
  return out


def _l1_xyz(gx, gy, gz, w, blk):
  n = gx.shape[0]
  c = w.shape[0]
  grid = n // blk
  wp = _pad8([w[:, 0], w[:, 1], w[:, 2]])
  return pl.pallas_call(
      _l1_xyz_body,
      grid=(grid,),
      in_specs=[pl.BlockSpec((blk, 1), lambda i: (i, 0))] * 3
      + [pl.BlockSpec((8, c), lambda i: (0, 0))],
      out_specs=[pl.BlockSpec((blk, c), lambda i: (i, 0)),
                 pl.BlockSpec((8, c), lambda i: (0, 0))],
      out_shape=[jax.ShapeDtypeStruct((n, c), jnp.float32),
                 jax.ShapeDtypeStruct((8, c), jnp.float32)],
  )(gx, gy, gz, wp)


def _l1_feat(gx, gy, gz, f, w, blk):
  n, cin = f.shape
  c = w.shape[0]
  grid = n // blk
  wp = _pad8([w[:, 0], w[:, 1], w[:, 2]])
  wf = jnp.transpose(w[:, 3:])
  return pl.pallas_call(
      _l1_feat_body,
      grid=(grid,),
      in_specs=[pl.BlockSpec((blk, 1), lambda i: (i, 0))] * 3
      + [pl.BlockSpec((blk, cin), lambda i: (i, 0)),
         pl.BlockSpec((8, c), lambda i: (0, 0)),
         pl.BlockSpec((cin, c), lambda i: (0, 0))],
      out_specs=[pl.BlockSpec((blk, c), lambda i: (i, 0)),
                 pl.BlockSpec((8, c), lambda i: (0, 0))],
      out_shape=[jax.ShapeDtypeStruct((n, c), jnp.float32),
                 jax.ShapeDtypeStruct((8, c), jnp.float32)],
  )(gx, gy, gz, f, wp, wf)


def _mid(x, st, g, b, w, blk):
  n, cin = x.shape
  c = w.shape[0]
  grid = n // blk
  gb = _pad8([g, b])
  wf = jnp.transpose(w)
  return pl.pallas_call(
      functools.partial(_mid_body, inv_n=1.0 / n),
      grid=(grid,),
      in_specs=[pl.BlockSpec((blk, cin), lambda i: (i, 0)),
                pl.BlockSpec((8, cin), lambda i: (0, 0)),
                pl.BlockSpec((8, cin), lambda i: (0, 0)),
                pl.BlockSpec((cin, c), lambda i: (0, 0))],
      out_specs=[pl.BlockSpec((blk, c), lambda i: (i, 0)),
                 pl.BlockSpec((8, c), lambda i: (0, 0))],
      out_shape=[jax.ShapeDtypeStruct((n, c), jnp.float32),
                 jax.ShapeDtypeStruct((8, c), jnp.float32)],
  )(x, st, gb, wf)


def _pool(x, st, g, b, grp, blk):
  n, c = x.shape
  grid = n // blk
  gb = _pad8([g, b])
  return pl.pallas_call(
      functools.partial(_pool_body, inv_n=1.0 / n, g=grp),
      grid=(grid,),
      in_specs=[pl.BlockSpec((blk, c), lambda i: (i, 0)),
                pl.BlockSpec((8, c), lambda i: (0, 0)),
                pl.BlockSpec((8, c), lambda i: (0, 0))],
      out_specs=pl.BlockSpec((blk // grp, c), lambda i: (i, 0)),
      out_shape=jax.ShapeDtypeStruct((n // grp, c), jnp.float32),
  )(x, st, gb)


def _cls_body(x_ref, w1_ref, b1_ref, w2_ref, b2_ref, w3_ref, b3_ref, o_ref):
  x = x_ref[...]
  x = jnp.dot(x, w1_ref[...], preferred_element_type=jnp.float32)
  x = jnp.maximum(x + b1_ref[0:1, :], 0.0)
  x = jnp.dot(x, w2_ref[...], preferred_element_type=jnp.float32)
  x = jnp.maximum(x + b2_ref[0:1, :], 0.0)
  x = jnp.dot(x, w3_ref[...], preferred_element_type=jnp.float32)
  o_ref[...] = x + b3_ref[0:1, :]


def _classifier(x, cls):
  (w1, b1), (w2, b2), (w3, b3) = cls
  n = x.shape[0]
  return pl.pallas_call(
      _cls_body,
      out_shape=jax.ShapeDtypeStruct((n, w3.shape[0]), jnp.float32),
  )(x, jnp.transpose(w1), _pad8([b1]), jnp.transpose(w2), _pad8([b2]),
    jnp.transpose(w3), _pad8([b3]))


def _shared_mlp_block(gx, gy, gz, feat, params, grp, blk):
  (w1, g1, b1), (w2, g2, b2), (w3, g3, b3) = params
  if feat is None:
    y, st = _l1_xyz(gx, gy, gz, w1, blk)
  else:
    y, st = _l1_feat(gx, gy, gz, feat, w1, blk)
  y, st2 = _mid(y, st, g1, b1, w2, blk)
  y, st3 = _mid(y, st2, g2, b2, w3, blk)
  return _pool(y, st3, g3, b3, grp, blk)


def kernel(input, mlp1, mlp2, mlp3, cls):
  B, N, _ = input.shape
  xs = input[:, :, 0]
  ys = input[:, :, 1]
  zs = input[:, :, 2]

  nx1, ny1, nz1 = _fps(xs, ys, zs, 512)
  gx1, gy1, gz1 = _ball_group(xs, ys, zs, nx1, ny1, nz1, 0.2, 32)
  n1 = B * 512 * 32
  f1 = _shared_mlp_block(gx1.reshape(n1, 1), gy1.reshape(n1, 1),
                         gz1.reshape(n1, 1), None, mlp1, 32, 4096)

  nx2, ny2, nz2 = _fps(nx1, ny1, nz1, 128)
  gx2, gy2, gz2, gi2 = _ball_group(nx1, ny1, nz1, nx2, ny2, nz2, 0.4, 64,
                                   base_stride=512)
  n2 = B * 128 * 64
  feat2 = _row_gather(f1, gi2.reshape(n2))
  f2 = _shared_mlp_block(gx2.reshape(n2, 1), gy2.reshape(n2, 1),
                         gz2.reshape(n2, 1), feat2, mlp2, 64, 4096)

  n3 = B * 128
  f3 = _shared_mlp_block(nx2.reshape(n3, 1), ny2.reshape(n3, 1),
                         nz2.reshape(n3, 1), f2, mlp3, 128, n3)

  return _classifier(f3, cls)


# speedup vs baseline: 1.0681x; 1.0681x over previous
import functools

import jax
import jax.numpy as jnp
from jax import lax
from jax.experimental import pallas as pl
from jax.experimental.pallas import tpu as pltpu
from jax.experimental.pallas import tpu_sc as plsc


def _fps_body(xs_ref, ys_ref, zs_ref, ox_ref, oy_ref, oz_ref, d_ref, *, npoint):
  B, N = xs_ref.shape
  xs = xs_ref[...]
  ys = ys_ref[...]
  zs = zs_ref[...]
  lane = lax.broadcasted_iota(jnp.int32, (B, N), 1)
  col = lax.broadcasted_iota(jnp.int32, (B, npoint), 1)
  d_ref[...] = jnp.full((B, N), 1e10, jnp.float32)
  ox_ref[...] = jnp.zeros((B, npoint), jnp.float32)
  oy_ref[...] = jnp.zeros((B, npoint), jnp.float32)
  oz_ref[...] = jnp.zeros((B, npoint), jnp.float32)

  def step(i, far):
    oh = (lane == far).astype(jnp.float32)
    cx = jnp.sum(xs * oh, axis=1, keepdims=True)
    cy = jnp.sum(ys * oh, axis=1, keepdims=True)
    cz = jnp.sum(zs * oh, axis=1, keepdims=True)
    coh = (col == i).astype(jnp.float32)
    ox_ref[...] += cx * coh
    oy_ref[...] += cy * coh
    oz_ref[...] += cz * coh
    dx = xs - cx
    dy = ys - cy
    dz = zs - cz
    d = dx * dx + dy * dy + dz * dz
    dm = jnp.minimum(d_ref[...], d)
    d_ref[...] = dm
    m = jnp.max(dm, axis=1, keepdims=True)
    far2 = jnp.min(jnp.where(dm == m, lane, N), axis=1, keepdims=True)
    return far2

  lax.fori_loop(0, npoint, step, jnp.zeros((B, 1), jnp.int32))


def _fps(xs, ys, zs, npoint):
  B, N = xs.shape
  out = jax.ShapeDtypeStruct((B, npoint), jnp.float32)
  return pl.pallas_call(
      functools.partial(_fps_body, npoint=npoint),
      out_shape=[out, out, out],
      scratch_shapes=[pltpu.VMEM((B, N), jnp.float32)],
  )(xs, ys, zs)


def _ball_group(xs, ys, zs, cx, cy, cz, radius, nsample, base_stride=None):
  B, N = xs.shape
  S = cx.shape[1]
  NW = 32
  WPB = NW // B
  SPW = S // WPB
  NCH = N // 16
  KCH = nsample // 16
  r2 = radius * radius
  want_idx = base_stride is not None

  out_type = [jax.ShapeDtypeStruct((B, S * nsample), jnp.float32)] * 3
  if want_idx:
    out_type.append(jax.ShapeDtypeStruct((B, S * nsample), jnp.int32))
  scratch = (
      [pltpu.VMEM((N,), jnp.float32)] * 3
      + [pltpu.VMEM((SPW,), jnp.float32)] * 3
      + [pltpu.VMEM((SPW * nsample,), jnp.float32)] * 3
      + [pltpu.VMEM((nsample,), jnp.int32), pltpu.VMEM((16,), jnp.int32),
         pltpu.VMEM((16,), jnp.int32)]
  )
  if want_idx:
    scratch.append(pltpu.VMEM((SPW * nsample,), jnp.int32))

  mesh = plsc.VectorSubcoreMesh(core_axis_name="c", subcore_axis_name="s")

  @functools.partial(
      pl.kernel, mesh=mesh, out_type=out_type, scratch_types=scratch,
      compiler_params=pltpu.CompilerParams(needs_layout_passes=False))
  def k(xs_h, ys_h, zs_h, cx_h, cy_h, cz_h, *rest):
    if want_idx:
      (ox_h, oy_h, oz_h, oi_h, xv, yv, zv, cxv, cyv, czv,
       gxb, gyb, gzb, idxb, cntb, zb, gib) = rest
    else:
      (ox_h, oy_h, oz_h, xv, yv, zv, cxv, cyv, czv,
       gxb, gyb, gzb, idxb, cntb, zb) = rest
    w = lax.axis_index("s") * 2 + lax.axis_index("c")
    b = w // WPB
    off = (w % WPB) * SPW
    pltpu.sync_copy(xs_h.at[b], xv)
    pltpu.sync_copy(ys_h.at[b], yv)
    pltpu.sync_copy(zs_h.at[b], zv)
    pltpu.sync_copy(cx_h.at[b, pl.ds(off, SPW)], cxv)
    pltpu.sync_copy(cy_h.at[b, pl.ds(off, SPW)], cyv)
    pltpu.sync_copy(cz_h.at[b, pl.ds(off, SPW)], czv)
    zb[pl.ds(0, 16)] = jnp.zeros((16,), jnp.int32)

    def per_cchunk(ci, carry):
      for k in range(16):
        sidx = ci * 16 + k

        def chunk(j, c2):
          base = pl.multiple_of(j * 16, 16)
          sp = jnp.full((16,), sidx, jnp.int32)
          c_xv = plsc.load_gather(cxv, [sp])
          c_yv = plsc.load_gather(cyv, [sp])
          c_zv = plsc.load_gather(czv, [sp])
          px = xv[pl.ds(base, 16)]
          py = yv[pl.ds(base, 16)]
          pz = zv[pl.ds(base, 16)]
          dx = px - c_xv
          dy = py - c_yv
          dz = pz - c_zv
          d2 = dx * dx + dy * dy + dz * dz
          m = d2 < r2
          mi = jnp.where(m, 1, 0)
          cnt_v = cntb[pl.ds(0, 16)]
          pos = cnt_v + plsc.cumsum(mi) - 1
          valid = jnp.logical_and(m, pos < nsample)
          iv = lax.iota(jnp.int32, 16) + base
          plsc.store_scatter(idxb, [pos], iv, mask=valid)
          cntb[pl.ds(0, 16)] = cnt_v + plsc.all_reduce_population_count(m)
          return c2

        cntb[pl.ds(0, 16)] = jnp.zeros((16,), jnp.int32)
        lax.fori_loop(0, NCH, chunk, 0)
        cnt_v = jnp.minimum(cntb[pl.ds(0, 16)], nsample)
        sp = jnp.full((16,), sidx, jnp.int32)
        c_xv = plsc.load_gather(cxv, [sp])
        c_yv = plsc.load_gather(cyv, [sp])
        c_zv = plsc.load_gather(czv, [sp])
        first = plsc.load_gather(idxb, [zb[pl.ds(0, 16)]])
        for kk in range(KCH):
          kb = kk * 16
          lanes = lax.iota(jnp.int32, 16) + kb
          v = idxb[pl.ds(kb, 16)]
          v = jnp.where(lanes < cnt_v, v, first)
          gx = plsc.load_gather(xv, [v]) - c_xv
          gy = plsc.load_gather(yv, [v]) - c_yv
          gz = plsc.load_gather(zv, [v]) - c_zv
          o = pl.multiple_of(sidx * nsample + kb, 16)
          gxb[pl.ds(o, 16)] = gx
          gyb[pl.ds(o, 16)] = gy
          gzb[pl.ds(o, 16)] = gz
          if want_idx:
            gib[pl.ds(o, 16)] = v + b * base_stride
      return carry

    lax.fori_loop(0, SPW // 16, per_cchunk, 0)
    dst = pl.ds(off * nsample, SPW * nsample)
    pltpu.sync_copy(gxb, ox_h.at[b, dst])
    pltpu.sync_copy(gyb, oy_h.at[b, dst])
    pltpu.sync_copy(gzb, oz_h.at[b, dst])
    if want_idx:
      pltpu.sync_copy(gib, oi_h.at[b, dst])

  return k(xs, ys, zs, cx, cy, cz)


def _row_gather(table, idx):
  V, D = table.shape
  R = idx.shape[0]
  NW = 32
  RPW = R // NW
  CH = 128
  NC = RPW // CH
  mesh = plsc.VectorSubcoreMesh(core_axis_name="c", subcore_axis_name="s")

  @functools.partial(
      pl.kernel, mesh=mesh,
      compiler_params=pltpu.CompilerParams(needs_layout_passes=False),
      out_type=jax.ShapeDtypeStruct((R, D), jnp.float32),
      scratch_types=[
          pltpu.VMEM((CH,), jnp.int32),
          pltpu.VMEM((CH, D), jnp.float32),
          pltpu.SemaphoreType.DMA,
      ],
  )
  def k(tab_h, idx_h, out_h, idxv, rowsv, sem):
    w = lax.axis_index("s") * 2 + lax.axis_index("c")
    basew = w * RPW

    def chunk(i, carry):
      o = basew + i * CH
      pltpu.sync_copy(idx_h.at[pl.ds(o, CH)], idxv)
      pltpu.async_copy(tab_h.at[idxv], rowsv, sem).wait()
      pltpu.sync_copy(rowsv, out_h.at[pl.ds(o, CH)])
      return carry

    lax.fori_loop(0, NC, chunk, 0)

  return k(table, idx)


def _b16(x):
  return x.astype(jnp.bfloat16).astype(jnp.float32)


def _stats_update(st_ref, y, is_first):
  @pl.when(is_first)
  def _():
    st_ref[...] = jnp.zeros_like(st_ref)
  s = jnp.sum(y, axis=0, keepdims=True)
  q = jnp.sum(y * y, axis=0, keepdims=True)
  pad = jnp.zeros((6, y.shape[1]), jnp.float32)
  st_ref[...] += jnp.concatenate([s, q, pad], axis=0)


def _bn_relu(x, st_ref, gb_ref, inv_n):
  mean = st_ref[0:1, :] * inv_n
  var = st_ref[1:2, :] * inv_n - mean * mean
  sc = gb_ref[0:1, :] * lax.rsqrt(var + 1e-5)
  sh = gb_ref[1:2, :] - mean * sc
  return jnp.maximum(x * sc + sh, 0.0)


def _l1_xyz_body(gx_ref, gy_ref, gz_ref, w_ref, y_ref, st_ref):
  y = (_b16(gx_ref[...]) * _b16(w_ref[0:1, :])
       + _b16(gy_ref[...]) * _b16(w_ref[1:2, :])
       + _b16(gz_ref[...]) * _b16(w_ref[2:3, :]))
  y_ref[...] = y
  _stats_update(st_ref, y, pl.program_id(0) == 0)


def _l1_feat_body(gx_ref, gy_ref, gz_ref, f_ref, w_ref, wf_ref, y_ref, st_ref):
  y = jnp.dot(f_ref[...], wf_ref[...], preferred_element_type=jnp.float32)
  y += (_b16(gx_ref[...]) * _b16(w_ref[0:1, :])
        + _b16(gy_ref[...]) * _b16(w_ref[1:2, :])
        + _b16(gz_ref[...]) * _b16(w_ref[2:3, :]))
  y_ref[...] = y
  _stats_update(st_ref, y, pl.program_id(0) == 0)


def _mid_body(x_ref, st_ref, gb_ref, wf_ref, y_ref, sto_ref, *, inv_n):
  z = _bn_relu(x_ref[...], st_ref, gb_ref, inv_n)
  y = jnp.dot(z, wf_ref[...], preferred_element_type=jnp.float32)
  y_ref[...] = y
  _stats_update(sto_ref, y, pl.program_id(0) == 0)


def _pool_body(x_ref, st_ref, gb_ref, o_ref, *, inv_n, g):
  R, C = x_ref.shape
  z = _bn_relu(x_ref[...], st_ref, gb_ref, inv_n)
  o_ref[...] = jnp.max(z.reshape(R // g, g, C), axis=1)


def _pad8(rows):
  c = rows[0].shape[0]
  out = jnp.zeros((8, c), jnp.float32)
  for i, r in enumerate(rows):
    out = out.at[i].set(r)
  return out


def _l1_xyz(gx, gy, gz, w, blk):
  n = gx.shape[0]
  c = w.shape[0]
  grid = n // blk
  wp = _pad8([w[:, 0], w[:, 1], w[:, 2]])
  return pl.pallas_call(
      _l1_xyz_body,
      grid=(grid,),
      in_specs=[pl.BlockSpec((blk, 1), lambda i: (i, 0))] * 3
      + [pl.BlockSpec((8, c), lambda i: (0, 0))],
      out_specs=[pl.BlockSpec((blk, c), lambda i: (i, 0)),
                 pl.BlockSpec((8, c), lambda i: (0, 0))],
      out_shape=[jax.ShapeDtypeStruct((n, c), jnp.float32),
                 jax.ShapeDtypeStruct((8, c), jnp.float32)],
  )(gx, gy, gz, wp)


def _l1_feat(gx, gy, gz, f, w, blk):
  n, cin = f.shape
  c = w.shape[0]
  grid = n // blk
  wp = _pad8([w[:, 0], w[:, 1], w[:, 2]])
  wf = jnp.transpose(w[:, 3:])
  return pl.pallas_call(
      _l1_feat_body,
      grid=(grid,),
      in_specs=[pl.BlockSpec((blk, 1), lambda i: (i, 0))] * 3
      + [pl.BlockSpec((blk, cin), lambda i: (i, 0)),
         pl.BlockSpec((8, c), lambda i: (0, 0)),
         pl.BlockSpec((cin, c), lambda i: (0, 0))],
      out_specs=[pl.BlockSpec((blk, c), lambda i: (i, 0)),
                 pl.BlockSpec((8, c), lambda i: (0, 0))],
      out_shape=[jax.ShapeDtypeStruct((n, c), jnp.float32),
                 jax.ShapeDtypeStruct((8, c), jnp.float32)],
  )(gx, gy, gz, f, wp, wf)


def _mid(x, st, g, b, w, blk):
  n, cin = x.shape
  c = w.shape[0]
  grid = n // blk
  gb = _pad8([g, b])
  wf = jnp.transpose(w)
  return pl.pallas_call(
      functools.partial(_mid_body, inv_n=1.0 / n),
      grid=(grid,),
      in_specs=[pl.BlockSpec((blk, cin), lambda i: (i, 0)),
                pl.BlockSpec((8, cin), lambda i: (0, 0)),
                pl.BlockSpec((8, cin), lambda i: (0, 0)),
                pl.BlockSpec((cin, c), lambda i: (0, 0))],
      out_specs=[pl.BlockSpec((blk, c), lambda i: (i, 0)),
                 pl.BlockSpec((8, c), lambda i: (0, 0))],
      out_shape=[jax.ShapeDtypeStruct((n, c), jnp.float32),
                 jax.ShapeDtypeStruct((8, c), jnp.float32)],
  )(x, st, gb, wf)


def _pool(x, st, g, b, grp, blk):
  n, c = x.shape
  grid = n // blk
  gb = _pad8([g, b])
  return pl.pallas_call(
      functools.partial(_pool_body, inv_n=1.0 / n, g=grp),
      grid=(grid,),
      in_specs=[pl.BlockSpec((blk, c), lambda i: (i, 0)),
                pl.BlockSpec((8, c), lambda i: (0, 0)),
                pl.BlockSpec((8, c), lambda i: (0, 0))],
      out_specs=pl.BlockSpec((blk // grp, c), lambda i: (i, 0)),
      out_shape=jax.ShapeDtypeStruct((n // grp, c), jnp.float32),
  )(x, st, gb)


def _cls_body(x_ref, w1_ref, b1_ref, w2_ref, b2_ref, w3_ref, b3_ref, o_ref):
  x = x_ref[...]
  x = jnp.dot(x, w1_ref[...], preferred_element_type=jnp.float32)
  x = jnp.maximum(x + b1_ref[0:1, :], 0.0)
  x = jnp.dot(x, w2_ref[...], preferred_element_type=jnp.float32)
  x = jnp.maximum(x + b2_ref[0:1, :], 0.0)
  x = jnp.dot(x, w3_ref[...], preferred_element_type=jnp.float32)
  o_ref[...] = x + b3_ref[0:1, :]


def _classifier(x, cls):
  (w1, b1), (w2, b2), (w3, b3) = cls
  n = x.shape[0]
  return pl.pallas_call(
      _cls_body,
      out_shape=jax.ShapeDtypeStruct((n, w3.shape[0]), jnp.float32),
  )(x, jnp.transpose(w1), _pad8([b1]), jnp.transpose(w2), _pad8([b2]),
    jnp.transpose(w3), _pad8([b3]))


def _shared_mlp_block(gx, gy, gz, feat, params, grp, blk):
  (w1, g1, b1), (w2, g2, b2), (w3, g3, b3) = params
  if feat is None:
    y, st = _l1_xyz(gx, gy, gz, w1, blk)
  else:
    y, st = _l1_feat(gx, gy, gz, feat, w1, blk)
  y, st2 = _mid(y, st, g1, b1, w2, blk)
  y, st3 = _mid(y, st2, g2, b2, w3, blk)
  return _pool(y, st3, g3, b3, grp, blk)


def kernel(input, mlp1, mlp2, mlp3, cls):
  B, N, _ = input.shape
  xs = input[:, :, 0]
  ys = input[:, :, 1]
  zs = input[:, :, 2]

  nx1, ny1, nz1 = _fps(xs, ys, zs, 512)
  gx1, gy1, gz1 = _ball_group(xs, ys, zs, nx1, ny1, nz1, 0.2, 32)
  n1 = B * 512 * 32
  f1 = _shared_mlp_block(gx1.reshape(n1, 1), gy1.reshape(n1, 1),
                         gz1.reshape(n1, 1), None, mlp1, 32, 4096)

  nx2, ny2, nz2 = _fps(nx1, ny1, nz1, 128)
  gx2, gy2, gz2, gi2 = _ball_group(nx1, ny1, nz1, nx2, ny2, nz2, 0.4, 64,
                                   base_stride=512)
  n2 = B * 128 * 64
  feat2 = _row_gather(f1, gi2.reshape(n2))
  f2 = _shared_mlp_block(gx2.reshape(n2, 1), gy2.reshape(n2, 1),
                         gz2.reshape(n2, 1), feat2, mlp2, 64, 4096)

  n3 = B * 128
  f3 = _shared_mlp_block(nx2.reshape(n3, 1), ny2.reshape(n3, 1),
                         nz2.reshape(n3, 1), f2, mlp3, 128, n3)

  return _classifier(f3, cls)
